# baseline (speedup 1.0000x reference)
import jax
import jax.numpy as jnp
from jax import lax
from jax.experimental import pallas as pl
from jax.experimental.pallas import tpu as pltpu
from jax.experimental.pallas import tpu_sc as plsc

VOCAB = 1_000_000
SEQ = 200
D = 64
BATCH = 4096

NC, NS = 2, 16
NW = NC * NS
L = 16
B_TOTAL = BATCH * SEQ
B_PER_W = B_TOTAL // NW
IBLK = 128
CHUNK = 256
KBLK = CHUNK // IBLK
N_CHUNKS = B_PER_W // CHUNK
POS_TILE = 6400
NFILL = POS_TILE // CHUNK


def _emb_kernel(idx_hbm, table_hbm, pos_hbm, out_hbm, idx_v, buf_v, out_v,
                in_s0, in_s1, g_s0, g_s1, o_s0, o_s1):
    in_sem = (in_s0, in_s1)
    g_sem = (g_s0, g_s1)
    out_sem = (o_s0, o_s1)
    wid = lax.axis_index("s") * NC + lax.axis_index("c")
    base = wid * B_PER_W

    def in_descs(g, s):
        row0 = base + g * CHUNK
        pos0 = lax.rem(g, NFILL) * CHUNK
        return (
            pltpu.make_async_copy(
                idx_hbm.at[pl.ds(row0, CHUNK)], idx_v.at[s], in_sem[s]),
            pltpu.make_async_copy(
                pos_hbm.at[pl.ds(pos0, CHUNK)], out_v.at[s], in_sem[s]),
        )

    def gather_desc(s, j):
        return pltpu.make_async_copy(
            table_hbm.at[idx_v.at[s, pl.ds(j * IBLK, IBLK)]],
            buf_v.at[j % 2],
            g_sem[s])

    def out_desc(g, s):
        row0 = base + g * CHUNK
        return pltpu.make_async_copy(
            out_v.at[s], out_hbm.at[pl.ds(row0, CHUNK)], out_sem[s])

    def add_low_halves(s, j):
        def rbody(q, _):
            for u in range(4):
                r = q * 4 + u
                for k in range(D // L):
                    plsc.addupdate(
                        out_v.at[s, j * IBLK + r, pl.ds(k * L, L)],
                        buf_v[j % 2, r, pl.ds(k * L, L)],
                    )
            return _
        lax.fori_loop(0, IBLK // 4, rbody, None)

    def process(g, s):
        for d in in_descs(g, s):
            d.wait()
        gather_desc(s, 0).start()
        o = 1 - s

        @pl.when(g > 0)
        def _():
            out_desc(g - 1, o).wait()

        @pl.when(g + 1 < N_CHUNKS)
        def _():
            for d in in_descs(g + 1, o):
                d.start()
        gather_desc(s, 0).wait()
        gather_desc(s, 1).start()
        add_low_halves(s, 0)
        gather_desc(s, 1).wait()
        add_low_halves(s, 1)
        out_desc(g, s).start()

    for d in in_descs(0, 0):
        d.start()

    def body(i, _):
        process(2 * i, 0)
        process(2 * i + 1, 1)
        return _

    lax.fori_loop(0, N_CHUNKS // 2, body, None)
    out_desc(N_CHUNKS - 1, 1).wait()


@jax.jit
def _embed(idx_flat, table_padded, pos_tiled):
    mesh = plsc.VectorSubcoreMesh(
        core_axis_name="c", subcore_axis_name="s", num_cores=NC, num_subcores=NS
    )
    fn = pl.kernel(
        _emb_kernel,
        out_type=jax.ShapeDtypeStruct((B_TOTAL, D), jnp.float32),
        mesh=mesh,
        scratch_types=[
            pltpu.VMEM((2, CHUNK), jnp.int32),
            pltpu.VMEM((2, IBLK, 2 * D), jnp.float32),
            pltpu.VMEM((2, CHUNK, D), jnp.float32),
            pltpu.SemaphoreType.DMA,
            pltpu.SemaphoreType.DMA,
            pltpu.SemaphoreType.DMA,
            pltpu.SemaphoreType.DMA,
            pltpu.SemaphoreType.DMA,
            pltpu.SemaphoreType.DMA,
        ],
    )
    return fn(idx_flat, table_padded, pos_tiled)


def kernel(inputs, token_table, pos_table):
    idx_flat = inputs.astype(jnp.int32).reshape(B_TOTAL)
    table_padded = jnp.pad(token_table.astype(jnp.float32), ((0, 0), (0, D)))
    pos_tiled = jnp.tile(pos_table.astype(jnp.float32), (POS_TILE // SEQ, 1))
    out = _embed(idx_flat, table_padded, pos_tiled)
    return out.reshape(BATCH, SEQ, D)


# speedup vs baseline: 1.0493x; 1.0493x over previous
import jax
import jax.numpy as jnp
from jax import lax
from jax.experimental import pallas as pl
from jax.experimental.pallas import tpu as pltpu
from jax.experimental.pallas import tpu_sc as plsc

VOCAB = 1_000_000
SEQ = 200
D = 64
BATCH = 4096

NC, NS = 2, 16
NW = NC * NS
L = 16
B_TOTAL = BATCH * SEQ
B_PER_W = B_TOTAL // NW
U = 128
N_UNITS = B_PER_W // U
POS_TILE = 3200
NFILL = POS_TILE // U
NQ = 4
NB = 2


def _emb_kernel(idx_hbm, table_hbm, pos_hbm, out_hbm, idx_v, buf_v, out_v,
                in_s0, in_s1, in_s2, in_s3, g_s0, g_s1,
                o_s0, o_s1, o_s2, o_s3):
    in_sem = (in_s0, in_s1, in_s2, in_s3)
    g_sem = (g_s0, g_s1)
    out_sem = (o_s0, o_s1, o_s2, o_s3)
    wid = lax.axis_index("s") * NC + lax.axis_index("c")
    base = wid * B_PER_W

    def in_descs(u, q):
        row0 = base + u * U
        pos0 = lax.rem(u, NFILL) * U
        return (
            pltpu.make_async_copy(
                idx_hbm.at[pl.ds(row0, U)], idx_v.at[q], in_sem[q]),
            pltpu.make_async_copy(
                pos_hbm.at[pl.ds(pos0, U)], out_v.at[q], in_sem[q]),
        )

    def gather_desc(q, a):
        return pltpu.make_async_copy(
            table_hbm.at[idx_v.at[q]], buf_v.at[a], g_sem[a])

    def out_desc(u, q):
        row0 = base + u * U
        return pltpu.make_async_copy(
            out_v.at[q], out_hbm.at[pl.ds(row0, U)], out_sem[q])

    def add_low_halves(q, a):
        def rbody(i, _):
            for uu in range(4):
                r = i * 4 + uu
                for k in range(D // L):
                    plsc.addupdate(
                        out_v.at[q, r, pl.ds(k * L, L)],
                        buf_v[a, r, pl.ds(k * L, L)],
                    )
            return _
        lax.fori_loop(0, U // 4, rbody, None)

    def start_in(u, q):
        for d in in_descs(u, q):
            d.start()

    def wait_in(u, q):
        for d in in_descs(u, q):
            d.wait()

    for up in range(3):
        start_in(up, up)
    wait_in(0, 0)
    gather_desc(0, 0).start()
    wait_in(1, 1)
    gather_desc(1, 1).start()

    def unit_step(u, q, a):
        gather_desc(q, a).wait()
        add_low_halves(q, a)
        out_desc(u, q).start()

        @pl.when(u + 2 < N_UNITS)
        def _():
            wait_in(u + 2, (q + 2) % NQ)
            gather_desc((q + 2) % NQ, a).start()

        @pl.when((u + 3 < N_UNITS) & (u >= 1))
        def _():
            out_desc(u - 1, (q + 3) % NQ).wait()
            start_in(u + 3, (q + 3) % NQ)

        @pl.when((u + 3 < N_UNITS) & (u < 1))
        def _():
            start_in(u + 3, (q + 3) % NQ)

    def body(i, _):
        for uu in range(NQ):
            u = i * NQ + uu
            unit_step(u, uu, uu % NB)
        return _

    lax.fori_loop(0, N_UNITS // NQ, body, None)
    for ut in range(N_UNITS - 4, N_UNITS):
        out_desc(ut, ut % NQ).wait()


@jax.jit
def _embed(idx_flat, table_padded, pos_tiled):
    mesh = plsc.VectorSubcoreMesh(
        core_axis_name="c", subcore_axis_name="s", num_cores=NC, num_subcores=NS
    )
    fn = pl.kernel(
        _emb_kernel,
        out_type=jax.ShapeDtypeStruct((B_TOTAL, D), jnp.float32),
        mesh=mesh,
        scratch_types=[
            pltpu.VMEM((NQ, U), jnp.int32),
            pltpu.VMEM((NB, U, 2 * D), jnp.float32),
            pltpu.VMEM((NQ, U, D), jnp.float32),
            pltpu.SemaphoreType.DMA,
            pltpu.SemaphoreType.DMA,
            pltpu.SemaphoreType.DMA,
            pltpu.SemaphoreType.DMA,
            pltpu.SemaphoreType.DMA,
            pltpu.SemaphoreType.DMA,
            pltpu.SemaphoreType.DMA,
            pltpu.SemaphoreType.DMA,
            pltpu.SemaphoreType.DMA,
            pltpu.SemaphoreType.DMA,
        ],
    )
    return fn(idx_flat, table_padded, pos_tiled)


def kernel(inputs, token_table, pos_table):
    idx_flat = inputs.astype(jnp.int32).reshape(B_TOTAL)
    table_padded = jnp.pad(token_table.astype(jnp.float32), ((0, 0), (0, D)))
    pos_tiled = jnp.tile(pos_table.astype(jnp.float32), (POS_TILE // SEQ, 1))
    out = _embed(idx_flat, table_padded, pos_tiled)
    return out.reshape(BATCH, SEQ, D)
